# baseline (speedup 1.0000x reference)
import functools

import jax
import jax.numpy as jnp
from jax import lax
from jax.experimental import pallas as pl
from jax.experimental.pallas import tpu as pltpu
from jax.experimental.pallas import tpu_sc as plsc

SEQ = 100000
ROW = 225
LANES = 16
WIN_BASE_MAX = SEQ - LANES


def _sc_lookup(pose_hbm, lin_hbm, tvec_hbm, out_hbm, tv_v, win_v, row_v):
    c = lax.axis_index("c")
    s = lax.axis_index("s")

    @pl.when(jnp.logical_and(c == 0, s == 0))
    def _():
        pltpu.sync_copy(tvec_hbm, tv_v)
        tv = tv_v[...]
        t = tv[0]
        i0 = (t * jnp.float32(SEQ - 1) + jnp.float32(0.5)).astype(jnp.int32)
        base = jnp.minimum(jnp.maximum(((i0 - 4) >> 3) << 3, 0), WIN_BASE_MAX)
        base = pl.multiple_of(base, 8)
        pltpu.sync_copy(lin_hbm.at[pl.ds(base, LANES)], win_v)
        d = jnp.abs(win_v[...] - tv)
        best_d = d[0]
        off = jnp.int32(0)
        for k in range(1, LANES):
            dk = d[k]
            better = dk < best_d
            best_d = jnp.where(better, dk, best_d)
            off = jnp.where(better, jnp.int32(k), off)
        i = base + off
        pltpu.sync_copy(pose_hbm.at[pl.ds(i, 1)], row_v)
        pltpu.sync_copy(row_v.at[0], out_hbm)


_mesh = plsc.VectorSubcoreMesh(core_axis_name="c", subcore_axis_name="s")

_lookup = functools.partial(
    pl.kernel,
    out_type=jax.ShapeDtypeStruct((ROW,), jnp.float32),
    mesh=_mesh,
    scratch_types=[
        pltpu.VMEM((LANES,), jnp.float32),
        pltpu.VMEM((LANES,), jnp.float32),
        pltpu.VMEM((1, ROW), jnp.float32),
    ],
)(_sc_lookup)


def kernel(pose_params, time_point):
    pose2d = pose_params.reshape(SEQ, ROW)
    lin = jnp.linspace(0, 1, SEQ)
    tvec = jnp.full((LANES,), time_point, dtype=jnp.float32)
    out = _lookup(pose2d, lin, tvec)
    return out.reshape(75, 3)
